# baseline (speedup 1.0000x reference)
import functools

import jax
import jax.numpy as jnp
import numpy as np
from jax import lax
from jax.experimental import pallas as pl
from jax.experimental.pallas import tpu as pltpu
from jax.experimental.pallas import tpu_sc as plsc

N = 10000
SEQ_LEN = 6
NUM_DIM = 2
RADIUS = 0.0226
LATENT = 128
NUM_MP_STEPS = 6
NUM_TYPES = 9
TYPE_EMB = 16
MAX_NEIGHBORS = 64

E = N * MAX_NEIGHBORS
_NTILES = N // 16
_NBLK = N // 16
_EB = 1024
_RB = _EB // MAX_NEIGHBORS
_EGRID = E // _EB
_NROWS = 1000
_NGRID = N // _NROWS


def _graph_sc(xs, ys, xsplat, ysplat):
    mesh = plsc.VectorSubcoreMesh(core_axis_name="c", subcore_axis_name="s")
    info = plsc.get_sparse_core_info()
    nc, ns = info.num_cores, info.num_subcores
    nw = nc * ns
    r2 = jnp.float32(RADIUS * RADIUS)

    @functools.partial(
        pl.kernel,
        mesh=mesh,
        compiler_params=pltpu.CompilerParams(needs_layout_passes=False),
        out_type=[
            jax.ShapeDtypeStruct((N * MAX_NEIGHBORS,), jnp.int32),
            jax.ShapeDtypeStruct((N * MAX_NEIGHBORS,), jnp.float32),
            jax.ShapeDtypeStruct((N * MAX_NEIGHBORS,), jnp.float32),
        ],
        scratch_types=[
            pltpu.VMEM((N,), jnp.float32),
            pltpu.VMEM((N,), jnp.float32),
            pltpu.VMEM((16, 16), jnp.float32),
            pltpu.VMEM((16, 16), jnp.float32),
            pltpu.VMEM((16 * MAX_NEIGHBORS,), jnp.int32),
            pltpu.VMEM((16 * MAX_NEIGHBORS,), jnp.float32),
            pltpu.VMEM((16 * MAX_NEIGHBORS,), jnp.float32),
        ],
    )
    def k(xs_hbm, ys_hbm, xsp_hbm, ysp_hbm, nbr_hbm, dx_hbm, dy_hbm,
          xs_v, ys_v, xsp_b, ysp_b, nbr_b, dx_b, dy_b):
        wid = lax.axis_index("s") * nc + lax.axis_index("c")
        pltpu.sync_copy(xs_hbm, xs_v)
        pltpu.sync_copy(ys_hbm, ys_v)
        lanes = lax.iota(jnp.int32, 16)

        def do_tile(kk, _):
            t = wid + nw * kk
            pltpu.sync_copy(xsp_hbm.at[pl.ds(t * 16, 16)], xsp_b)
            pltpu.sync_copy(ysp_hbm.at[pl.ds(t * 16, 16)], ysp_b)

            for rr in range(16):
                xi = xsp_b[rr, pl.ds(0, 16)]
                yi = ysp_b[rr, pl.ds(0, 16)]
                for k4 in range(MAX_NEIGHBORS // 16):
                    base = rr * MAX_NEIGHBORS + k4 * 16
                    nbr_b[pl.ds(base, 16)] = jnp.full((16,), N, jnp.int32)
                    dx_b[pl.ds(base, 16)] = jnp.zeros((16,), jnp.float32)
                    dy_b[pl.ds(base, 16)] = jnp.zeros((16,), jnp.float32)

                def blk(jb, cnt):
                    xj = xs_v[pl.ds(jb * 16, 16)]
                    yj = ys_v[pl.ds(jb * 16, 16)]
                    dx = xj - xi
                    dy = yj - yi
                    d2 = dx * dx + dy * dy
                    mask = d2 <= r2
                    mi = jnp.where(mask, 1, 0).astype(jnp.int32)
                    csum = plsc.cumsum(mi)
                    idx = cnt + csum - 1
                    smask = mask & (idx < MAX_NEIGHBORS)
                    sidx = rr * MAX_NEIGHBORS + idx
                    col = jb * 16 + lanes
                    plsc.store_scatter(nbr_b, [sidx], col, mask=smask)
                    plsc.store_scatter(dx_b, [sidx], dx, mask=smask)
                    plsc.store_scatter(dy_b, [sidx], dy, mask=smask)
                    return cnt + csum[15]

                lax.fori_loop(0, _NBLK, blk, jnp.int32(0))

            ob = t * 16 * MAX_NEIGHBORS
            pltpu.sync_copy(nbr_b, nbr_hbm.at[pl.ds(ob, 16 * MAX_NEIGHBORS)])
            pltpu.sync_copy(dx_b, dx_hbm.at[pl.ds(ob, 16 * MAX_NEIGHBORS)])
            pltpu.sync_copy(dy_b, dy_hbm.at[pl.ds(ob, 16 * MAX_NEIGHBORS)])
            return 0

        nt = (_NTILES - wid + nw - 1) // nw
        lax.fori_loop(0, nt, do_tile, 0)

    return k(xs, ys, xsplat, ysplat)


_GCH = 128
_GNBUF = 5
_EPW = 20480
_EPAD = _EPW * 32


def _gather_sc(table, idx_pad):
    mesh = plsc.VectorSubcoreMesh(core_axis_name="c", subcore_axis_name="s")
    info = plsc.get_sparse_core_info()
    nc, ns = info.num_cores, info.num_subcores
    nchunks = _EPW // _GCH

    @functools.partial(
        pl.kernel,
        mesh=mesh,
        compiler_params=pltpu.CompilerParams(needs_layout_passes=False),
        out_type=jax.ShapeDtypeStruct((_EPAD, LATENT), jnp.float32),
        scratch_types=[
            pltpu.VMEM((_EPW,), jnp.int32),
        ] + [pltpu.VMEM((_GCH, LATENT), jnp.float32)] * _GNBUF
          + [pltpu.SemaphoreType.DMA] * _GNBUF,
    )
    def gk(tab_hbm, idx_hbm, out_hbm, idx_v, r0, r1, r2b, r3, r4,
           s0, s1, s2, s3, s4):
        rows = (r0, r1, r2b, r3, r4)
        sems = (s0, s1, s2, s3, s4)
        wid = lax.axis_index("s") * nc + lax.axis_index("c")
        base0 = wid * _EPW
        pltpu.sync_copy(idx_hbm.at[pl.ds(base0, _EPW)], idx_v)

        for b in range(_GNBUF):
            pltpu.async_copy(tab_hbm.at[idx_v.at[pl.ds(b * _GCH, _GCH)]],
                             rows[b], sems[b])

        def grp(g, _):
            for b in range(_GNBUF):
                c = g * _GNBUF + b
                pltpu.make_async_copy(
                    tab_hbm.at[idx_v.at[pl.ds(0, _GCH)]], rows[b], sems[b]
                ).wait()
                pltpu.sync_copy(rows[b],
                                out_hbm.at[pl.ds(base0 + c * _GCH, _GCH)])

                @pl.when(c + _GNBUF < nchunks)
                def _():
                    pltpu.async_copy(
                        tab_hbm.at[idx_v.at[pl.ds((c + _GNBUF) * _GCH, _GCH)]],
                        rows[b], sems[b])
            return 0

        lax.fori_loop(0, nchunks // _GNBUF, grp, 0)

    return gk(table, idx_pad)


def _lnv(x, g, beta):
    mu = jnp.mean(x, axis=-1, keepdims=True)
    d = x - mu
    var = jnp.mean(d * d, axis=-1, keepdims=True)
    return d / jnp.sqrt(var + 1e-6) * g + beta


def _node_enc_kernel(fv_ref, dist_ref, oh_ref,
                     w1v, w1d, w1t, b1, w2, b2, w3, b3, g, beta, w1s, w1r,
                     nl_ref, ps_ref, pr_ref):
    h = jnp.dot(fv_ref[...], w1v[...], preferred_element_type=jnp.float32)
    h = h + jnp.dot(dist_ref[...], w1d[...], preferred_element_type=jnp.float32)
    h = h + jnp.dot(oh_ref[...], w1t[...], preferred_element_type=jnp.float32)
    h = jnp.maximum(h + b1[...], 0.0)
    h = jnp.maximum(jnp.dot(h, w2[...], preferred_element_type=jnp.float32) + b2[...], 0.0)
    h = jnp.dot(h, w3[...], preferred_element_type=jnp.float32) + b3[...]
    nl = _lnv(h, g[...], beta[...])
    nl_ref[...] = nl
    ps_ref[...] = jnp.dot(nl, w1s[...], preferred_element_type=jnp.float32)
    pr_ref[...] = jnp.dot(nl, w1r[...], preferred_element_type=jnp.float32)


def _edge_enc_kernel(dx_ref, dy_ref,
                     w1x, w1y, w1n, b1, w2, b2, w3, b3, g, beta,
                     el_ref):
    rx = dx_ref[...] * (1.0 / RADIUS)
    ry = dy_ref[...] * (1.0 / RADIUS)
    rd = jnp.sqrt(rx * rx + ry * ry)
    h = jnp.dot(rx, w1x[...], preferred_element_type=jnp.float32)
    h = h + jnp.dot(ry, w1y[...], preferred_element_type=jnp.float32)
    h = h + jnp.dot(rd, w1n[...], preferred_element_type=jnp.float32)
    h = jnp.maximum(h + b1[...], 0.0)
    h = jnp.maximum(jnp.dot(h, w2[...], preferred_element_type=jnp.float32) + b2[...], 0.0)
    h = jnp.dot(h, w3[...], preferred_element_type=jnp.float32) + b3[...]
    el_ref[...] = _lnv(h, g[...], beta[...])


def _mp_kernel(el_ref, gs_ref, nbr_ref, pr_ref, nl_ref, bc_ref,
               w1e, b1, w2, b2, w3, b3, g, beta,
               v1a, v1b, c1, u2, c2, u3, c3, gn, betan,
               w1sn, w1rn,
               enew_ref, nlnew_ref, psn_ref, prn_ref):
    el = el_ref[...]
    bc = bc_ref[...]
    prb = jnp.dot(bc, pr_ref[...], preferred_element_type=jnp.float32)
    h = jnp.dot(el, w1e[...], preferred_element_type=jnp.float32)
    h = jnp.maximum(h + gs_ref[...] + prb + b1[...], 0.0)
    h = jnp.maximum(jnp.dot(h, w2[...], preferred_element_type=jnp.float32) + b2[...], 0.0)
    h = jnp.dot(h, w3[...], preferred_element_type=jnp.float32) + b3[...]
    enew = _lnv(h, g[...], beta[...]) + el
    enew_ref[...] = enew

    validf = (nbr_ref[...] < N).astype(jnp.float32)
    agg = lax.dot_general(bc, enew * validf, (((0,), (0,)), ((), ())),
                          preferred_element_type=jnp.float32)
    nl = nl_ref[...]
    h2 = jnp.dot(nl, v1a[...], preferred_element_type=jnp.float32)
    h2 = h2 + jnp.dot(agg, v1b[...], preferred_element_type=jnp.float32)
    h2 = jnp.maximum(h2 + c1[...], 0.0)
    h2 = jnp.maximum(jnp.dot(h2, u2[...], preferred_element_type=jnp.float32) + c2[...], 0.0)
    h2 = jnp.dot(h2, u3[...], preferred_element_type=jnp.float32) + c3[...]
    nlnew = _lnv(h2, gn[...], betan[...]) + nl
    nlnew_ref[...] = nlnew
    psn_ref[...] = jnp.dot(nlnew, w1sn[...], preferred_element_type=jnp.float32)
    prn_ref[...] = jnp.dot(nlnew, w1rn[...], preferred_element_type=jnp.float32)


def _dec_kernel(nl_ref, mr_ref, rv_ref,
                d1, e1, d2, e2, d3, e3, astd, amean,
                out_ref):
    h = jnp.maximum(jnp.dot(nl_ref[...], d1[...], preferred_element_type=jnp.float32) + e1[...], 0.0)
    h = jnp.maximum(jnp.dot(h, d2[...], preferred_element_type=jnp.float32) + e2[...], 0.0)
    y = jnp.dot(h, d3[...], preferred_element_type=jnp.float32) + e3[...]
    acc = y * astd[...] + amean[...]
    out_ref[...] = mr_ref[...] + rv_ref[...] + acc


def _full(shape):
    return pl.BlockSpec(shape, lambda i: (0, 0))


def kernel(position_sequence, n_particles_per_example, particle_types, params):
    f32 = jnp.float32
    most_recent = position_sequence[:, -1]
    xs = most_recent[:, 0]
    ys = most_recent[:, 1]
    xsplat = jnp.broadcast_to(xs[:, None], (N, 16))
    ysplat = jnp.broadcast_to(ys[:, None], (N, 16))
    nbrs, dxs, dys = _graph_sc(xs, ys, xsplat, ysplat)

    senders = jnp.where(nbrs < N, nbrs, 0).astype(jnp.int32)
    senders_pad = jnp.concatenate(
        [senders, jnp.zeros(_EPAD - E, jnp.int32)])

    vel = position_sequence[:, 1:] - position_sequence[:, :-1]
    nvel = (vel - params["vel_mean"]) / params["vel_std"]
    flat_vel = nvel.reshape(N, (SEQ_LEN - 1) * NUM_DIM)
    boundaries = jnp.array([[0.0, 1.0], [0.0, 1.0]], f32)
    dlow = most_recent - boundaries[:, 0]
    dup = boundaries[:, 1] - most_recent
    dist = jnp.clip(jnp.concatenate([dlow, dup], axis=1) / RADIUS, -1.0, 1.0)
    oh = (particle_types[:, None] == jnp.arange(16)[None, :]).astype(f32)

    ne = params["node_enc"]
    w1 = ne["W"][0]
    nv = (SEQ_LEN - 1) * NUM_DIM
    w1v, w1d = w1[:nv], w1[nv:nv + 4]
    temb_pad = jnp.zeros((16, TYPE_EMB), f32).at[:NUM_TYPES].set(params["type_emb"])
    w1t = temb_pad @ w1[nv + 4:]
    proc = params["proc"]
    r2 = lambda b: b.reshape(1, -1)

    def estep_w(s):
        st = proc[s]["edge"]
        ew1 = st["W"][0]
        return (ew1[:LATENT], ew1[LATENT:2 * LATENT], ew1[2 * LATENT:])

    w1s0, w1r0 = estep_w(0)[1], estep_w(0)[2]

    nspec = pl.BlockSpec((_NROWS, LATENT), lambda i: (i, 0))
    nl0, ps, pr = pl.pallas_call(
        _node_enc_kernel,
        grid=(_NGRID,),
        in_specs=[
            pl.BlockSpec((_NROWS, nv), lambda i: (i, 0)),
            pl.BlockSpec((_NROWS, 4), lambda i: (i, 0)),
            pl.BlockSpec((_NROWS, 16), lambda i: (i, 0)),
            _full((nv, LATENT)), _full((4, LATENT)), _full((16, LATENT)),
            _full((1, LATENT)), _full((LATENT, LATENT)), _full((1, LATENT)),
            _full((LATENT, LATENT)), _full((1, LATENT)), _full((1, LATENT)),
            _full((1, LATENT)), _full((LATENT, LATENT)), _full((LATENT, LATENT)),
        ],
        out_specs=[nspec, nspec, nspec],
        out_shape=[jax.ShapeDtypeStruct((N, LATENT), f32)] * 3,
    )(flat_vel, dist, oh,
      w1v, w1d, w1t, r2(ne["b"][0]), ne["W"][1], r2(ne["b"][1]),
      ne["W"][2], r2(ne["b"][2]), r2(ne["g"]), r2(ne["beta"]), w1s0, w1r0)

    ee = params["edge_enc"]
    ew1 = ee["W"][0]
    espec = pl.BlockSpec((_EB, LATENT), lambda i: (i, 0))
    cspec = pl.BlockSpec((_EB, 1), lambda i: (i, 0))
    edge_lat = pl.pallas_call(
        _edge_enc_kernel,
        grid=(_EGRID,),
        in_specs=[
            cspec, cspec,
            _full((1, LATENT)), _full((1, LATENT)), _full((1, LATENT)),
            _full((1, LATENT)), _full((LATENT, LATENT)), _full((1, LATENT)),
            _full((LATENT, LATENT)), _full((1, LATENT)),
            _full((1, LATENT)), _full((1, LATENT)),
        ],
        out_specs=espec,
        out_shape=jax.ShapeDtypeStruct((E, LATENT), f32),
    )(dxs.reshape(E, 1), dys.reshape(E, 1),
      ew1[0:1], ew1[1:2], ew1[2:3], r2(ee["b"][0]), ee["W"][1], r2(ee["b"][1]),
      ee["W"][2], r2(ee["b"][2]), r2(ee["g"]), r2(ee["beta"]))

    bc = jnp.asarray(np.repeat(np.eye(_RB, dtype=np.float32), MAX_NEIGHBORS, axis=0))
    nbr_col = nbrs.reshape(E, 1)
    zeros128 = jnp.zeros((LATENT, LATENT), f32)
    nl = nl0
    rspec = pl.BlockSpec((_RB, LATENT), lambda i: (i, 0))

    for s in range(NUM_MP_STEPS):
        gs = _gather_sc(ps, senders_pad)
        est = proc[s]["edge"]
        nst = proc[s]["node"]
        w1e = estep_w(s)[0]
        if s + 1 < NUM_MP_STEPS:
            w1sn, w1rn = estep_w(s + 1)[1], estep_w(s + 1)[2]
        else:
            w1sn, w1rn = zeros128, zeros128
        nv1 = nst["W"][0]
        edge_lat, nl, ps, pr = pl.pallas_call(
            _mp_kernel,
            grid=(_EGRID,),
            in_specs=[
                espec, espec,
                pl.BlockSpec((_EB, 1), lambda i: (i, 0)),
                rspec, rspec,
                _full((_EB, _RB)),
                _full((LATENT, LATENT)), _full((1, LATENT)),
                _full((LATENT, LATENT)), _full((1, LATENT)),
                _full((LATENT, LATENT)), _full((1, LATENT)),
                _full((1, LATENT)), _full((1, LATENT)),
                _full((LATENT, LATENT)), _full((LATENT, LATENT)),
                _full((1, LATENT)),
                _full((LATENT, LATENT)), _full((1, LATENT)),
                _full((LATENT, LATENT)), _full((1, LATENT)),
                _full((1, LATENT)), _full((1, LATENT)),
                _full((LATENT, LATENT)), _full((LATENT, LATENT)),
            ],
            out_specs=[espec, rspec, rspec, rspec],
            out_shape=[
                jax.ShapeDtypeStruct((E, LATENT), f32),
                jax.ShapeDtypeStruct((N, LATENT), f32),
                jax.ShapeDtypeStruct((N, LATENT), f32),
                jax.ShapeDtypeStruct((N, LATENT), f32),
            ],
        )(edge_lat, gs, nbr_col, pr, nl, bc,
          w1e, r2(est["b"][0]), est["W"][1], r2(est["b"][1]),
          est["W"][2], r2(est["b"][2]), r2(est["g"]), r2(est["beta"]),
          nv1[:LATENT], nv1[LATENT:], r2(nst["b"][0]),
          nst["W"][1], r2(nst["b"][1]), nst["W"][2], r2(nst["b"][2]),
          r2(nst["g"]), r2(nst["beta"]),
          w1sn, w1rn)

    dec = params["dec"]
    recent_vel = position_sequence[:, -1] - position_sequence[:, -2]
    out = pl.pallas_call(
        _dec_kernel,
        grid=(_NGRID,),
        in_specs=[
            pl.BlockSpec((_NROWS, LATENT), lambda i: (i, 0)),
            pl.BlockSpec((_NROWS, NUM_DIM), lambda i: (i, 0)),
            pl.BlockSpec((_NROWS, NUM_DIM), lambda i: (i, 0)),
            _full((LATENT, LATENT)), _full((1, LATENT)),
            _full((LATENT, LATENT)), _full((1, LATENT)),
            _full((LATENT, NUM_DIM)), _full((1, NUM_DIM)),
            _full((1, NUM_DIM)), _full((1, NUM_DIM)),
        ],
        out_specs=pl.BlockSpec((_NROWS, NUM_DIM), lambda i: (i, 0)),
        out_shape=jax.ShapeDtypeStruct((N, NUM_DIM), f32),
    )(nl, most_recent, recent_vel,
      dec["W"][0], r2(dec["b"][0]), dec["W"][1], r2(dec["b"][1]),
      dec["W"][2], r2(dec["b"][2]),
      r2(params["acc_std"]), r2(params["acc_mean"]))
    return out


# speedup vs baseline: 9.7891x; 9.7891x over previous
import functools

import jax
import jax.numpy as jnp
import numpy as np
from jax import lax
from jax.experimental import pallas as pl
from jax.experimental.pallas import tpu as pltpu
from jax.experimental.pallas import tpu_sc as plsc

N = 10000
SEQ_LEN = 6
NUM_DIM = 2
RADIUS = 0.0226
LATENT = 128
NUM_MP_STEPS = 6
NUM_TYPES = 9
TYPE_EMB = 16
MAX_NEIGHBORS = 64

E = N * MAX_NEIGHBORS
_NTILES = N // 16
_NBLK = N // 16
_EB = 1024
_RB = _EB // MAX_NEIGHBORS
_EGRID = E // _EB
_NROWS = 1000
_NGRID = N // _NROWS


def _graph_sc(xs, ys, xsplat, ysplat):
    mesh = plsc.VectorSubcoreMesh(core_axis_name="c", subcore_axis_name="s")
    info = plsc.get_sparse_core_info()
    nc, ns = info.num_cores, info.num_subcores
    nw = nc * ns
    r2 = jnp.float32(RADIUS * RADIUS)

    @functools.partial(
        pl.kernel,
        mesh=mesh,
        compiler_params=pltpu.CompilerParams(needs_layout_passes=False),
        out_type=[
            jax.ShapeDtypeStruct((N * MAX_NEIGHBORS,), jnp.int32),
            jax.ShapeDtypeStruct((N * MAX_NEIGHBORS,), jnp.float32),
            jax.ShapeDtypeStruct((N * MAX_NEIGHBORS,), jnp.float32),
        ],
        scratch_types=[
            pltpu.VMEM((N,), jnp.float32),
            pltpu.VMEM((N,), jnp.float32),
            pltpu.VMEM((16, 16), jnp.float32),
            pltpu.VMEM((16, 16), jnp.float32),
            pltpu.VMEM((16 * MAX_NEIGHBORS,), jnp.int32),
            pltpu.VMEM((16 * MAX_NEIGHBORS,), jnp.float32),
            pltpu.VMEM((16 * MAX_NEIGHBORS,), jnp.float32),
        ],
    )
    def k(xs_hbm, ys_hbm, xsp_hbm, ysp_hbm, nbr_hbm, dx_hbm, dy_hbm,
          xs_v, ys_v, xsp_b, ysp_b, nbr_b, dx_b, dy_b):
        wid = lax.axis_index("s") * nc + lax.axis_index("c")
        pltpu.sync_copy(xs_hbm, xs_v)
        pltpu.sync_copy(ys_hbm, ys_v)
        lanes = lax.iota(jnp.int32, 16)

        def do_tile(kk, _):
            t = wid + nw * kk
            pltpu.sync_copy(xsp_hbm.at[pl.ds(t * 16, 16)], xsp_b)
            pltpu.sync_copy(ysp_hbm.at[pl.ds(t * 16, 16)], ysp_b)

            for rr in range(16):
                xi = xsp_b[rr, pl.ds(0, 16)]
                yi = ysp_b[rr, pl.ds(0, 16)]
                for k4 in range(MAX_NEIGHBORS // 16):
                    base = rr * MAX_NEIGHBORS + k4 * 16
                    nbr_b[pl.ds(base, 16)] = jnp.full((16,), N, jnp.int32)
                    dx_b[pl.ds(base, 16)] = jnp.zeros((16,), jnp.float32)
                    dy_b[pl.ds(base, 16)] = jnp.zeros((16,), jnp.float32)

                def blk(jb, cnt):
                    xj = xs_v[pl.ds(jb * 16, 16)]
                    yj = ys_v[pl.ds(jb * 16, 16)]
                    dx = xj - xi
                    dy = yj - yi
                    d2 = dx * dx + dy * dy
                    mask = d2 <= r2
                    mi = jnp.where(mask, 1, 0).astype(jnp.int32)
                    csum = plsc.cumsum(mi)
                    idx = cnt + csum - 1
                    smask = mask & (idx < MAX_NEIGHBORS)
                    sidx = rr * MAX_NEIGHBORS + idx
                    col = jb * 16 + lanes
                    plsc.store_scatter(nbr_b, [sidx], col, mask=smask)
                    plsc.store_scatter(dx_b, [sidx], dx, mask=smask)
                    plsc.store_scatter(dy_b, [sidx], dy, mask=smask)
                    return cnt + csum[15]

                lax.fori_loop(0, _NBLK, blk, jnp.int32(0))

            ob = t * 16 * MAX_NEIGHBORS
            pltpu.sync_copy(nbr_b, nbr_hbm.at[pl.ds(ob, 16 * MAX_NEIGHBORS)])
            pltpu.sync_copy(dx_b, dx_hbm.at[pl.ds(ob, 16 * MAX_NEIGHBORS)])
            pltpu.sync_copy(dy_b, dy_hbm.at[pl.ds(ob, 16 * MAX_NEIGHBORS)])
            return 0

        nt = (_NTILES - wid + nw - 1) // nw
        lax.fori_loop(0, nt, do_tile, 0)

    return k(xs, ys, xsplat, ysplat)


_GCH = 64
_GNBUF = 4
_IDXH = 10240
_EPW = 20480
_EPAD = _EPW * 32


def _gather_sc(table, idx_pad):
    mesh = plsc.VectorSubcoreMesh(core_axis_name="c", subcore_axis_name="s")
    info = plsc.get_sparse_core_info()
    nc, ns = info.num_cores, info.num_subcores
    nchunks = _IDXH // _GCH

    @functools.partial(
        pl.kernel,
        mesh=mesh,
        compiler_params=pltpu.CompilerParams(needs_layout_passes=False),
        out_type=jax.ShapeDtypeStruct((_EPAD, LATENT), jnp.float32),
        scratch_types=[
            pltpu.VMEM((_IDXH,), jnp.int32),
            pltpu.VMEM_SHARED((N, LATENT), jnp.float32),
        ] + [pltpu.VMEM((_GCH, LATENT), jnp.float32)] * _GNBUF
          + [pltpu.SemaphoreType.DMA] * _GNBUF,
    )
    def gk(tab_hbm, idx_hbm, out_hbm, idx_v, tab_sp, r0, r1, r2b, r3,
           s0, s1, s2, s3):
        rows = (r0, r1, r2b, r3)
        sems = (s0, s1, s2, s3)
        wid = lax.axis_index("s") * nc + lax.axis_index("c")
        sid = lax.axis_index("s")

        for j in range(10):
            q = sid * 10 + j

            @pl.when(q < 156)
            def _():
                pltpu.sync_copy(tab_hbm.at[pl.ds(q * 64, 64)], r0)
                pltpu.sync_copy(r0, tab_sp.at[pl.ds(q * 64, 64)])

        @pl.when(sid == 15)
        def _():
            bounce = r0.at[pl.ds(0, 16)]
            pltpu.sync_copy(tab_hbm.at[pl.ds(156 * 64, 16)], bounce)
            pltpu.sync_copy(bounce, tab_sp.at[pl.ds(156 * 64, 16)])

        plsc.subcore_barrier()

        base0 = wid * _EPW
        for ph in range(2):
            pb = base0 + ph * _IDXH
            pltpu.sync_copy(idx_hbm.at[pl.ds(pb, _IDXH)], idx_v)
            for b in range(_GNBUF):
                pltpu.async_copy(tab_sp.at[idx_v.at[pl.ds(b * _GCH, _GCH)]],
                                 rows[b], sems[b])

            def grp(g, _):
                for b in range(_GNBUF):
                    c = g * _GNBUF + b
                    pltpu.make_async_copy(
                        tab_sp.at[idx_v.at[pl.ds(0, _GCH)]], rows[b], sems[b]
                    ).wait()
                    pltpu.sync_copy(rows[b],
                                    out_hbm.at[pl.ds(pb + c * _GCH, _GCH)])

                    @pl.when(c + _GNBUF < nchunks)
                    def _():
                        pltpu.async_copy(
                            tab_sp.at[idx_v.at[pl.ds((c + _GNBUF) * _GCH, _GCH)]],
                            rows[b], sems[b])
                return 0

            lax.fori_loop(0, nchunks // _GNBUF, grp, 0)

    return gk(table, idx_pad)


def _lnv(x, g, beta):
    mu = jnp.mean(x, axis=-1, keepdims=True)
    d = x - mu
    var = jnp.mean(d * d, axis=-1, keepdims=True)
    return d / jnp.sqrt(var + 1e-6) * g + beta


def _node_enc_kernel(fv_ref, dist_ref, oh_ref,
                     w1v, w1d, w1t, b1, w2, b2, w3, b3, g, beta, w1s, w1r,
                     nl_ref, ps_ref, pr_ref):
    h = jnp.dot(fv_ref[...], w1v[...], preferred_element_type=jnp.float32)
    h = h + jnp.dot(dist_ref[...], w1d[...], preferred_element_type=jnp.float32)
    h = h + jnp.dot(oh_ref[...], w1t[...], preferred_element_type=jnp.float32)
    h = jnp.maximum(h + b1[...], 0.0)
    h = jnp.maximum(jnp.dot(h, w2[...], preferred_element_type=jnp.float32) + b2[...], 0.0)
    h = jnp.dot(h, w3[...], preferred_element_type=jnp.float32) + b3[...]
    nl = _lnv(h, g[...], beta[...])
    nl_ref[...] = nl
    ps_ref[...] = jnp.dot(nl, w1s[...], preferred_element_type=jnp.float32)
    pr_ref[...] = jnp.dot(nl, w1r[...], preferred_element_type=jnp.float32)


def _edge_enc_kernel(dx_ref, dy_ref,
                     w1x, w1y, w1n, b1, w2, b2, w3, b3, g, beta,
                     el_ref):
    rx = dx_ref[...] * (1.0 / RADIUS)
    ry = dy_ref[...] * (1.0 / RADIUS)
    rd = jnp.sqrt(rx * rx + ry * ry)
    h = jnp.dot(rx, w1x[...], preferred_element_type=jnp.float32)
    h = h + jnp.dot(ry, w1y[...], preferred_element_type=jnp.float32)
    h = h + jnp.dot(rd, w1n[...], preferred_element_type=jnp.float32)
    h = jnp.maximum(h + b1[...], 0.0)
    h = jnp.maximum(jnp.dot(h, w2[...], preferred_element_type=jnp.float32) + b2[...], 0.0)
    h = jnp.dot(h, w3[...], preferred_element_type=jnp.float32) + b3[...]
    el_ref[...] = _lnv(h, g[...], beta[...])


def _mp_kernel(el_ref, gs_ref, nbr_ref, pr_ref, nl_ref, bc_ref,
               w1e, b1, w2, b2, w3, b3, g, beta,
               v1a, v1b, c1, u2, c2, u3, c3, gn, betan,
               w1sn, w1rn,
               enew_ref, nlnew_ref, psn_ref, prn_ref):
    el = el_ref[...]
    bc = bc_ref[...]
    prb = jnp.dot(bc, pr_ref[...], preferred_element_type=jnp.float32)
    h = jnp.dot(el, w1e[...], preferred_element_type=jnp.float32)
    h = jnp.maximum(h + gs_ref[...] + prb + b1[...], 0.0)
    h = jnp.maximum(jnp.dot(h, w2[...], preferred_element_type=jnp.float32) + b2[...], 0.0)
    h = jnp.dot(h, w3[...], preferred_element_type=jnp.float32) + b3[...]
    enew = _lnv(h, g[...], beta[...]) + el
    enew_ref[...] = enew

    validf = (nbr_ref[...] < N).astype(jnp.float32)
    agg = lax.dot_general(bc, enew * validf, (((0,), (0,)), ((), ())),
                          preferred_element_type=jnp.float32)
    nl = nl_ref[...]
    h2 = jnp.dot(nl, v1a[...], preferred_element_type=jnp.float32)
    h2 = h2 + jnp.dot(agg, v1b[...], preferred_element_type=jnp.float32)
    h2 = jnp.maximum(h2 + c1[...], 0.0)
    h2 = jnp.maximum(jnp.dot(h2, u2[...], preferred_element_type=jnp.float32) + c2[...], 0.0)
    h2 = jnp.dot(h2, u3[...], preferred_element_type=jnp.float32) + c3[...]
    nlnew = _lnv(h2, gn[...], betan[...]) + nl
    nlnew_ref[...] = nlnew
    psn_ref[...] = jnp.dot(nlnew, w1sn[...], preferred_element_type=jnp.float32)
    prn_ref[...] = jnp.dot(nlnew, w1rn[...], preferred_element_type=jnp.float32)


def _dec_kernel(nl_ref, mr_ref, rv_ref,
                d1, e1, d2, e2, d3, e3, astd, amean,
                out_ref):
    h = jnp.maximum(jnp.dot(nl_ref[...], d1[...], preferred_element_type=jnp.float32) + e1[...], 0.0)
    h = jnp.maximum(jnp.dot(h, d2[...], preferred_element_type=jnp.float32) + e2[...], 0.0)
    y = jnp.dot(h, d3[...], preferred_element_type=jnp.float32) + e3[...]
    acc = y * astd[...] + amean[...]
    out_ref[...] = mr_ref[...] + rv_ref[...] + acc


def _full(shape):
    return pl.BlockSpec(shape, lambda i: (0, 0))


def kernel(position_sequence, n_particles_per_example, particle_types, params):
    f32 = jnp.float32
    most_recent = position_sequence[:, -1]
    xs = most_recent[:, 0]
    ys = most_recent[:, 1]
    xsplat = jnp.broadcast_to(xs[:, None], (N, 16))
    ysplat = jnp.broadcast_to(ys[:, None], (N, 16))
    nbrs, dxs, dys = _graph_sc(xs, ys, xsplat, ysplat)

    senders = jnp.where(nbrs < N, nbrs, 0).astype(jnp.int32)
    senders_pad = jnp.concatenate(
        [senders, jnp.zeros(_EPAD - E, jnp.int32)])

    vel = position_sequence[:, 1:] - position_sequence[:, :-1]
    nvel = (vel - params["vel_mean"]) / params["vel_std"]
    flat_vel = nvel.reshape(N, (SEQ_LEN - 1) * NUM_DIM)
    boundaries = jnp.array([[0.0, 1.0], [0.0, 1.0]], f32)
    dlow = most_recent - boundaries[:, 0]
    dup = boundaries[:, 1] - most_recent
    dist = jnp.clip(jnp.concatenate([dlow, dup], axis=1) / RADIUS, -1.0, 1.0)
    oh = (particle_types[:, None] == jnp.arange(16)[None, :]).astype(f32)

    ne = params["node_enc"]
    w1 = ne["W"][0]
    nv = (SEQ_LEN - 1) * NUM_DIM
    w1v, w1d = w1[:nv], w1[nv:nv + 4]
    temb_pad = jnp.zeros((16, TYPE_EMB), f32).at[:NUM_TYPES].set(params["type_emb"])
    w1t = temb_pad @ w1[nv + 4:]
    proc = params["proc"]
    r2 = lambda b: b.reshape(1, -1)

    def estep_w(s):
        st = proc[s]["edge"]
        ew1 = st["W"][0]
        return (ew1[:LATENT], ew1[LATENT:2 * LATENT], ew1[2 * LATENT:])

    w1s0, w1r0 = estep_w(0)[1], estep_w(0)[2]

    nspec = pl.BlockSpec((_NROWS, LATENT), lambda i: (i, 0))
    nl0, ps, pr = pl.pallas_call(
        _node_enc_kernel,
        grid=(_NGRID,),
        in_specs=[
            pl.BlockSpec((_NROWS, nv), lambda i: (i, 0)),
            pl.BlockSpec((_NROWS, 4), lambda i: (i, 0)),
            pl.BlockSpec((_NROWS, 16), lambda i: (i, 0)),
            _full((nv, LATENT)), _full((4, LATENT)), _full((16, LATENT)),
            _full((1, LATENT)), _full((LATENT, LATENT)), _full((1, LATENT)),
            _full((LATENT, LATENT)), _full((1, LATENT)), _full((1, LATENT)),
            _full((1, LATENT)), _full((LATENT, LATENT)), _full((LATENT, LATENT)),
        ],
        out_specs=[nspec, nspec, nspec],
        out_shape=[jax.ShapeDtypeStruct((N, LATENT), f32)] * 3,
    )(flat_vel, dist, oh,
      w1v, w1d, w1t, r2(ne["b"][0]), ne["W"][1], r2(ne["b"][1]),
      ne["W"][2], r2(ne["b"][2]), r2(ne["g"]), r2(ne["beta"]), w1s0, w1r0)

    ee = params["edge_enc"]
    ew1 = ee["W"][0]
    espec = pl.BlockSpec((_EB, LATENT), lambda i: (i, 0))
    cspec = pl.BlockSpec((_EB, 1), lambda i: (i, 0))
    edge_lat = pl.pallas_call(
        _edge_enc_kernel,
        grid=(_EGRID,),
        in_specs=[
            cspec, cspec,
            _full((1, LATENT)), _full((1, LATENT)), _full((1, LATENT)),
            _full((1, LATENT)), _full((LATENT, LATENT)), _full((1, LATENT)),
            _full((LATENT, LATENT)), _full((1, LATENT)),
            _full((1, LATENT)), _full((1, LATENT)),
        ],
        out_specs=espec,
        out_shape=jax.ShapeDtypeStruct((E, LATENT), f32),
    )(dxs.reshape(E, 1), dys.reshape(E, 1),
      ew1[0:1], ew1[1:2], ew1[2:3], r2(ee["b"][0]), ee["W"][1], r2(ee["b"][1]),
      ee["W"][2], r2(ee["b"][2]), r2(ee["g"]), r2(ee["beta"]))

    bc = jnp.asarray(np.repeat(np.eye(_RB, dtype=np.float32), MAX_NEIGHBORS, axis=0))
    nbr_col = nbrs.reshape(E, 1)
    zeros128 = jnp.zeros((LATENT, LATENT), f32)
    nl = nl0
    rspec = pl.BlockSpec((_RB, LATENT), lambda i: (i, 0))

    for s in range(NUM_MP_STEPS):
        gs = _gather_sc(ps, senders_pad)
        est = proc[s]["edge"]
        nst = proc[s]["node"]
        w1e = estep_w(s)[0]
        if s + 1 < NUM_MP_STEPS:
            w1sn, w1rn = estep_w(s + 1)[1], estep_w(s + 1)[2]
        else:
            w1sn, w1rn = zeros128, zeros128
        nv1 = nst["W"][0]
        edge_lat, nl, ps, pr = pl.pallas_call(
            _mp_kernel,
            grid=(_EGRID,),
            in_specs=[
                espec, espec,
                pl.BlockSpec((_EB, 1), lambda i: (i, 0)),
                rspec, rspec,
                _full((_EB, _RB)),
                _full((LATENT, LATENT)), _full((1, LATENT)),
                _full((LATENT, LATENT)), _full((1, LATENT)),
                _full((LATENT, LATENT)), _full((1, LATENT)),
                _full((1, LATENT)), _full((1, LATENT)),
                _full((LATENT, LATENT)), _full((LATENT, LATENT)),
                _full((1, LATENT)),
                _full((LATENT, LATENT)), _full((1, LATENT)),
                _full((LATENT, LATENT)), _full((1, LATENT)),
                _full((1, LATENT)), _full((1, LATENT)),
                _full((LATENT, LATENT)), _full((LATENT, LATENT)),
            ],
            out_specs=[espec, rspec, rspec, rspec],
            out_shape=[
                jax.ShapeDtypeStruct((E, LATENT), f32),
                jax.ShapeDtypeStruct((N, LATENT), f32),
                jax.ShapeDtypeStruct((N, LATENT), f32),
                jax.ShapeDtypeStruct((N, LATENT), f32),
            ],
        )(edge_lat, gs, nbr_col, pr, nl, bc,
          w1e, r2(est["b"][0]), est["W"][1], r2(est["b"][1]),
          est["W"][2], r2(est["b"][2]), r2(est["g"]), r2(est["beta"]),
          nv1[:LATENT], nv1[LATENT:], r2(nst["b"][0]),
          nst["W"][1], r2(nst["b"][1]), nst["W"][2], r2(nst["b"][2]),
          r2(nst["g"]), r2(nst["beta"]),
          w1sn, w1rn)

    dec = params["dec"]
    recent_vel = position_sequence[:, -1] - position_sequence[:, -2]
    out = pl.pallas_call(
        _dec_kernel,
        grid=(_NGRID,),
        in_specs=[
            pl.BlockSpec((_NROWS, LATENT), lambda i: (i, 0)),
            pl.BlockSpec((_NROWS, NUM_DIM), lambda i: (i, 0)),
            pl.BlockSpec((_NROWS, NUM_DIM), lambda i: (i, 0)),
            _full((LATENT, LATENT)), _full((1, LATENT)),
            _full((LATENT, LATENT)), _full((1, LATENT)),
            _full((LATENT, NUM_DIM)), _full((1, NUM_DIM)),
            _full((1, NUM_DIM)), _full((1, NUM_DIM)),
        ],
        out_specs=pl.BlockSpec((_NROWS, NUM_DIM), lambda i: (i, 0)),
        out_shape=jax.ShapeDtypeStruct((N, NUM_DIM), f32),
    )(nl, most_recent, recent_vel,
      dec["W"][0], r2(dec["b"][0]), dec["W"][1], r2(dec["b"][1]),
      dec["W"][2], r2(dec["b"][2]),
      r2(params["acc_std"]), r2(params["acc_mean"]))
    return out
